# baseline (speedup 1.0000x reference)
import jax
import jax.numpy as jnp
from jax import lax
from jax.experimental import pallas as pl
from jax.experimental.pallas import tpu as pltpu
from jax.experimental.pallas import tpu_sc as plsc

BATCH = 16384
FEAT = 812
TOTAL = BATCH * FEAT

NC = 2
NS = 16
NW = NC * NS
SPAN = TOTAL // NW

MROWS = 4
MLEN = MROWS * FEAT
PERIODS = 8
CHUNK = PERIODS * MLEN
NCHUNK = SPAN // CHUNK
NVM = MLEN // 16


def _make_mask():
    key = jax.random.key(42)
    k_prob, k_cat = jax.random.split(key)
    prob = jax.random.uniform(k_prob, (1, 1), minval=0.0, maxval=1.0,
                              dtype=jnp.float32)
    prob_total = jnp.concatenate([prob, 1.0 - prob], axis=1)
    a = jax.random.categorical(k_cat, jnp.log(prob_total), axis=-1,
                               shape=(1, FEAT))
    return a.astype(jnp.float32).reshape(FEAT)


def _sc_body(x_hbm, m_hbm, o1_hbm, o2_hbm, xv, o1v, o2v, mv):
    c = lax.axis_index("c")
    s = lax.axis_index("s")
    wid = s * NC + c
    base = wid * SPAN

    pltpu.sync_copy(m_hbm, mv)

    def chunk_body(g, carry):
        e0 = base + g * CHUNK
        pltpu.sync_copy(x_hbm.at[pl.ds(e0, CHUNK)], xv)

        def period_body(p, carry2):
            p0 = p * MLEN
            for j in range(NVM):
                vx = xv[pl.ds(p0 + 16 * j, 16)]
                vm = mv[pl.ds(16 * j, 16)]
                v1 = vx * vm
                o1v[pl.ds(p0 + 16 * j, 16)] = v1
                o2v[pl.ds(p0 + 16 * j, 16)] = vx - v1
            return carry2

        lax.fori_loop(0, PERIODS, period_body, 0)
        pltpu.sync_copy(o1v, o1_hbm.at[pl.ds(e0, CHUNK)])
        pltpu.sync_copy(o2v, o2_hbm.at[pl.ds(e0, CHUNK)])
        return carry

    lax.fori_loop(0, NCHUNK, chunk_body, 0)


def kernel(inputs):
    x = inputs
    m4 = jnp.tile(_make_mask(), MROWS)
    mesh = plsc.VectorSubcoreMesh(core_axis_name="c", subcore_axis_name="s")
    f = pl.kernel(
        _sc_body,
        mesh=mesh,
        out_type=[
            jax.ShapeDtypeStruct((TOTAL,), jnp.float32),
            jax.ShapeDtypeStruct((TOTAL,), jnp.float32),
        ],
        scratch_types=[
            pltpu.VMEM((CHUNK,), jnp.float32),
            pltpu.VMEM((CHUNK,), jnp.float32),
            pltpu.VMEM((CHUNK,), jnp.float32),
            pltpu.VMEM((MLEN,), jnp.float32),
        ],
    )
    out1, out2 = f(x.reshape(TOTAL), m4)
    return (out1.reshape(BATCH, FEAT), out2.reshape(BATCH, FEAT))
